# baseline (speedup 1.0000x reference)
import functools

import jax
import jax.numpy as jnp
from jax import lax
from jax.experimental import pallas as pl
from jax.experimental.pallas import tpu as pltpu
from jax.experimental.pallas import tpu_sc as plsc

_B = 16
_P = 512
_D = 512
_F = 2048
_CHUNK = 32
_WIN = _CHUNK + 1
_NCHUNKS = _F // _CHUNK
_TCHUNKS = _NCHUNKS // 2


@functools.partial(
    pl.kernel,
    out_type=[
        jax.ShapeDtypeStruct((_B * _F * _D,), jnp.float32),
        jax.ShapeDtypeStruct((_B * _F,), jnp.int32),
    ],
    mesh=plsc.VectorSubcoreMesh(core_axis_name="c", subcore_axis_name="s"),
    compiler_params=pltpu.CompilerParams(needs_layout_passes=False),
    scratch_types=[
        pltpu.VMEM((_P,), jnp.int32),
        pltpu.VMEM((_F,), jnp.int32),
        pltpu.VMEM((_F,), jnp.int32),
        pltpu.VMEM((_D,), jnp.float32),
        pltpu.VMEM((_CHUNK * _D,), jnp.float32),
        pltpu.VMEM((_WIN * _D,), jnp.float32),
        pltpu.VMEM((_WIN * _D,), jnp.float32),
        pltpu.VMEM((_CHUNK * _D,), jnp.float32),
        pltpu.VMEM((_CHUNK * _D,), jnp.float32),
        pltpu.SemaphoreType.DMA,
        pltpu.SemaphoreType.DMA,
        pltpu.SemaphoreType.DMA,
        pltpu.SemaphoreType.DMA,
    ],
)
def _length_regulate(x_hbm, dur_hbm, out_hbm, val_hbm,
                     dur_v, sbuf, val_v, trow_v, tail_v,
                     src0, src1, obf0, obf1, rs0, rs1, ws0, ws1):
    srcs = (src0, src1)
    obfs = (obf0, obf1)
    rsems = (rs0, rs1)
    wsems = (ws0, ws1)
    b = lax.axis_index("s")
    half = lax.axis_index("c")

    pltpu.sync_copy(dur_hbm.at[b], dur_v)

    zero = jnp.zeros((16,), jnp.int32)
    one = jnp.ones((16,), jnp.int32)

    def zero_body(i, carry):
        sbuf[pl.ds(i * 16, 16)] = zero
        return carry
    lax.fori_loop(0, _F // 16, zero_body, 0)

    def scat_body(i, carry):
        v = dur_v[pl.ds(i * 16, 16)]
        cum = plsc.cumsum(v) + carry
        plsc.store_scatter(sbuf, [cum], one, mask=cum < _F)
        return carry + jnp.sum(v)
    lax.fori_loop(0, _P // 16, scat_body, jnp.int32(0))

    def scan_body(i, carry):
        v = sbuf[pl.ds(i * 16, 16)]
        val_v[pl.ds(i * 16, 16)] = plsc.cumsum(v) + carry
        return carry + jnp.sum(v)
    lax.fori_loop(0, _F // 16, scan_body, jnp.int32(0))

    @pl.when(half == 0)
    def _():
        pltpu.sync_copy(val_v, val_hbm.at[pl.ds(b * _F, _F)])

    iota = lax.iota(jnp.int32, 16)
    last = jnp.where(iota == 15, jnp.int32(1), jnp.int32(0))

    def lane15(off):
        return jnp.sum(val_v[pl.ds(off, 16)] * last)

    def window_base(g):
        base = jnp.where(g > 0, lane15(jnp.maximum(32 * g - 16, 0)), 0)
        return jnp.minimum(base + b * _P, _B * _P - _WIN)

    def chunk_const(g):
        return (jnp.sum(sbuf[pl.ds(32 * g, 16)])
                + jnp.sum(sbuf[pl.ds(32 * g + 16, 16)])) == 0

    trow = jnp.minimum(lane15(_F - 16) + b * _P, _B * _P - 1)
    pltpu.sync_copy(x_hbm.at[pl.ds(trow * _D, _D)], trow_v)

    def tail_body(t, carry):
        for kk in range(_D // 16):
            tail_v[pl.ds(t * _D + kk * 16, 16)] = trow_v[pl.ds(kk * 16, 16)]
        return carry
    lax.fori_loop(0, _CHUNK, tail_body, 0)

    def start_read(k, slot):
        g = 2 * k + half

        @pl.when(jnp.logical_not(chunk_const(g)))
        def _():
            pltpu.async_copy(
                x_hbm.at[pl.ds(window_base(g) * _D, _WIN * _D)],
                srcs[slot], rsems[slot])

    def wait_read(slot):
        pltpu.make_async_copy(
            x_hbm.at[pl.ds(0, _WIN * _D)], srcs[slot], rsems[slot]).wait()

    def wait_write(slot):
        pltpu.make_async_copy(
            obfs[slot], out_hbm.at[pl.ds(0, _CHUNK * _D)], wsems[slot]).wait()

    def expand(k, slot):
        g = 2 * k + half
        base_abs = window_base(g)
        for grp in range(2):
            v = val_v[pl.ds(32 * g + 16 * grp, 16)]
            rel = jnp.minimum(v + b * _P, _B * _P - 1) - base_abs
            is0 = rel * _D
            io0 = (iota + 16 * grp) * _D

            def copy_body(t, carry):
                is_ = is0 + t * 16
                io_ = io0 + t * 16
                for kk in range(16):
                    w = plsc.load_gather(srcs[slot], [is_ + kk])
                    plsc.store_scatter(obfs[slot], [io_ + kk], w)
                return carry
            lax.fori_loop(0, _D // 16, copy_body, 0)

    def do_chunk(k, slot, first):
        g = 2 * k + half
        if not first:
            wait_write(slot)
        cnd = chunk_const(g)
        dst = out_hbm.at[pl.ds((b * _F + 32 * g) * _D, _CHUNK * _D)]

        @pl.when(jnp.logical_not(cnd))
        def _():
            wait_read(slot)
            expand(k, slot)
            pltpu.async_copy(obfs[slot], dst, wsems[slot])

        @pl.when(cnd)
        def _():
            pltpu.async_copy(tail_v, dst, wsems[slot])

    start_read(0, 0)
    start_read(1, 1)
    do_chunk(0, 0, True)
    start_read(2, 0)
    do_chunk(1, 1, True)
    start_read(3, 1)

    def main_body(m, carry):
        for slot in range(2):
            k = 2 * m + slot
            do_chunk(k, slot, False)

            @pl.when(k + 2 < _TCHUNKS)
            def _():
                start_read(k + 2, slot)
        return carry
    lax.fori_loop(1, _TCHUNKS // 2, main_body, 0)

    wait_write(0)
    wait_write(1)


def kernel(x, durations):
    B, P, D = x.shape
    x_flat = x.reshape(B * P * D)
    out_flat, val = _length_regulate(x_flat, durations)
    out = out_flat.reshape(B, _F, D)
    val = val.reshape(B, _F)
    return out, val == (P - 1)


# speedup vs baseline: 4.8920x; 4.8920x over previous
import functools

import jax
import jax.numpy as jnp
from jax import lax
from jax.experimental import pallas as pl
from jax.experimental.pallas import tpu as pltpu
from jax.experimental.pallas import tpu_sc as plsc

_B = 16
_P = 512
_D = 512
_F = 2048
_CHUNK = 32
_NCHUNKS = _F // _CHUNK
_TCHUNKS = _NCHUNKS // 2
_NBUF = 3


@functools.partial(
    pl.kernel,
    out_type=[
        jax.ShapeDtypeStruct((_B * _F, _D), jnp.float32),
        jax.ShapeDtypeStruct((_B * _F,), jnp.int32),
    ],
    mesh=plsc.VectorSubcoreMesh(core_axis_name="c", subcore_axis_name="s"),
    compiler_params=pltpu.CompilerParams(needs_layout_passes=False),
    scratch_types=[
        pltpu.VMEM((_P,), jnp.int32),
        pltpu.VMEM((_F,), jnp.int32),
        pltpu.VMEM((_F,), jnp.int32),
        pltpu.VMEM((_F,), jnp.int32),
        pltpu.VMEM((_CHUNK,), jnp.int32),
        pltpu.VMEM((_CHUNK, _D), jnp.float32),
        pltpu.VMEM((_CHUNK, _D), jnp.float32),
        pltpu.VMEM((_CHUNK, _D), jnp.float32),
        pltpu.VMEM((_CHUNK, _D), jnp.float32),
        pltpu.SemaphoreType.DMA,
        pltpu.SemaphoreType.DMA,
        pltpu.SemaphoreType.DMA,
        pltpu.SemaphoreType.DMA,
        pltpu.SemaphoreType.DMA,
        pltpu.SemaphoreType.DMA,
    ],
)
def _length_regulate(x_hbm, dur_hbm, out_hbm, val_hbm,
                     dur_v, sbuf, val_v, row_v, tidx_v, tail_v,
                     obf0, obf1, obf2, rs0, rs1, rs2, ws0, ws1, ws2):
    obfs = (obf0, obf1, obf2)
    rsems = (rs0, rs1, rs2)
    wsems = (ws0, ws1, ws2)
    b = lax.axis_index("s")
    half = lax.axis_index("c")

    pltpu.sync_copy(dur_hbm.at[b], dur_v)

    zero = jnp.zeros((16,), jnp.int32)
    one = jnp.ones((16,), jnp.int32)

    def zero_body(i, carry):
        sbuf[pl.ds(i * 16, 16)] = zero
        return carry
    lax.fori_loop(0, _F // 16, zero_body, 0)

    def scat_body(i, carry):
        v = dur_v[pl.ds(i * 16, 16)]
        cum = plsc.cumsum(v) + carry
        plsc.store_scatter(sbuf, [cum], one, mask=cum < _F)
        return carry + jnp.sum(v)
    lax.fori_loop(0, _P // 16, scat_body, jnp.int32(0))

    def scan_body(i, carry):
        v = sbuf[pl.ds(i * 16, 16)]
        s = plsc.cumsum(v) + carry
        val_v[pl.ds(i * 16, 16)] = s
        row_v[pl.ds(i * 16, 16)] = jnp.minimum(s + b * _P, _B * _P - 1)
        return carry + jnp.sum(v)
    lax.fori_loop(0, _F // 16, scan_body, jnp.int32(0))

    @pl.when(half == 0)
    def _():
        pltpu.sync_copy(val_v, val_hbm.at[pl.ds(b * _F, _F)])

    iota = lax.iota(jnp.int32, 16)
    last = jnp.where(iota == 15, jnp.int32(1), jnp.int32(0))

    def chunk_const(g):
        return (jnp.sum(sbuf[pl.ds(32 * g, 16)])
                + jnp.sum(sbuf[pl.ds(32 * g + 16, 16)])) == 0

    trow = jnp.sum(row_v[pl.ds(_F - 16, 16)] * last)
    tidx_v[pl.ds(0, 16)] = jnp.full((16,), trow, jnp.int32)
    tidx_v[pl.ds(16, 16)] = jnp.full((16,), trow, jnp.int32)
    pltpu.sync_copy(x_hbm.at[tidx_v], tail_v)

    def start_fetch(k, slot):
        g = 2 * k + half

        @pl.when(jnp.logical_not(chunk_const(g)))
        def _():
            pltpu.async_copy(
                x_hbm.at[row_v.at[pl.ds(32 * g, _CHUNK)]],
                obfs[slot], rsems[slot])

    def wait_fetch(slot):
        pltpu.make_async_copy(
            x_hbm.at[row_v.at[pl.ds(0, _CHUNK)]],
            obfs[slot], rsems[slot]).wait()

    def wait_write(slot):
        pltpu.make_async_copy(
            obfs[slot], out_hbm.at[pl.ds(0, _CHUNK)], wsems[slot]).wait()

    def do_chunk(k, slot):
        g = 2 * k + half
        cnd = chunk_const(g)
        dst = out_hbm.at[pl.ds(b * _F + 32 * g, _CHUNK)]

        @pl.when(jnp.logical_not(cnd))
        def _():
            wait_fetch(slot)
            pltpu.async_copy(obfs[slot], dst, wsems[slot])

        @pl.when(cnd)
        def _():
            pltpu.async_copy(tail_v, dst, wsems[slot])

    for k in range(_NBUF):
        start_fetch(k, k)
    for k in range(_TCHUNKS):
        slot = k % _NBUF
        do_chunk(k, slot)
        if k + _NBUF < _TCHUNKS:
            wait_write(slot)
            start_fetch(k + _NBUF, slot)
    for k in range(_TCHUNKS - _NBUF, _TCHUNKS):
        wait_write(k % _NBUF)


def kernel(x, durations):
    B, P, D = x.shape
    x_flat = x.reshape(B * P, D)
    out_flat, val = _length_regulate(x_flat, durations)
    out = out_flat.reshape(B, _F, D)
    val = val.reshape(B, _F)
    return out, val == (P - 1)
